# baseline (speedup 1.0000x reference)
import jax
import jax.numpy as jnp
from jax.experimental import pallas as pl
from jax.experimental.pallas import tpu as pltpu

_NCLS = 6
_NUM_ATOMS = 5
_CP = 8
_TILE = 512
_STAGE0 = 1
_SPACING = 1


def _gelu(v):
    return 0.5 * v * (1.0 + jax.lax.erf(v * (2.0 ** -0.5)))


def _router(cls_tok, gd_tok):
    t_count = cls_tok.shape[0]
    logits = jnp.sum(cls_tok * gd_tok, axis=1, keepdims=True)
    choose_left = logits >= 0.0
    t = jax.lax.broadcasted_iota(jnp.int32, (t_count, 1), 0)
    n = t % _CP
    valid = n < _NCLS
    l = n // 2
    r = 3 + (n % 2)
    src = jnp.where(valid & choose_left, l, jnp.where(valid, r, -1))
    dst = jnp.where(valid & choose_left, r, jnp.where(valid, l, -1))
    p = jax.nn.sigmoid(logits)
    w = jnp.where(choose_left, p, 1.0 - p)
    return src, dst, w


_NCHUNK = 8


def _expert_copies(src3, dst, sem, e):
    rows = dst.shape[0] // _NCHUNK
    return [
        pltpu.make_async_copy(
            src3.at[e, pl.ds(c * rows, rows), :],
            dst.at[pl.ds(c * rows, rows), :], sem)
        for c in range(_NCHUNK)
    ]


def _start_expert(src3, dst, sem, e):
    for cp in _expert_copies(src3, dst, sem, e):
        cp.start()


def _wait_expert(src3, dst, sem, e):
    for cp in _expert_copies(src3, dst, sem, e):
        cp.wait()


def _expert_bias(e, b_ref):
    oh = (jax.lax.broadcasted_iota(jnp.int32, (1, _NUM_ATOMS), 1)
          == e).astype(jnp.float32)
    return jax.lax.dot_general(
        oh, b_ref[...], (((1,), (0,)), ((), ())),
        preferred_element_type=jnp.float32)


def _fused_kernel(x_ref, w1_ref, b1_ref, w2_ref, b2_ref,
                  cls_ref, gd_ref, ain_ref, binb_ref, aout_ref, boutb_ref,
                  out_ref,
                  w1b_ref, w2b_ref, ein_ref, eout_ref, hid_ref, acc_ref,
                  sin, sout):
    i = pl.program_id(0)
    n_steps = pl.num_programs(0)

    @pl.when(i == 0)
    def _():
        w1b_ref[...] = w1_ref[...].astype(jnp.bfloat16)
        w2b_ref[...] = w2_ref[...].astype(jnp.bfloat16)
        _start_expert(ain_ref, ein_ref, sin, 0)
        _start_expert(aout_ref, eout_ref, sout, 0)

    xb = x_ref[...].astype(jnp.bfloat16)
    h = jax.lax.dot_general(
        xb, w1b_ref[...], (((1,), (1,)), ((), ())),
        preferred_element_type=jnp.float32)
    h = _gelu((h + b1_ref[...]).astype(jnp.bfloat16))
    res = jax.lax.dot_general(
        h, w2b_ref[...], (((1,), (1,)), ((), ())),
        preferred_element_type=jnp.float32) + b2_ref[...]
    out_ref[...] = res

    stage = (i - _STAGE0) // _SPACING
    is_stage = ((i - _STAGE0) >= 0) & ((i - _STAGE0) % _SPACING == 0) & (
        stage < 2 * _NUM_ATOMS)

    @pl.when(is_stage & (stage < _NUM_ATOMS))
    def _():
        e = stage
        _wait_expert(ain_ref, ein_ref, sin, e)
        src, _, _ = _router(cls_ref[...], gd_ref[...])
        val = jax.lax.dot_general(
            cls_ref[...], ein_ref[...], (((1,), (1,)), ((), ())),
            preferred_element_type=jnp.float32)
        val = _gelu(val + _expert_bias(e, binb_ref))
        mask = src == e

        @pl.when(e == 0)
        def _():
            hid_ref[...] = jnp.where(mask, val, 0.0)

        @pl.when(e != 0)
        def _():
            hid_ref[...] = jnp.where(mask, val, hid_ref[...])

        @pl.when(e < _NUM_ATOMS - 1)
        def _():
            _start_expert(ain_ref, ein_ref, sin, e + 1)


    @pl.when(is_stage & (stage >= _NUM_ATOMS))
    def _():
        e = stage - _NUM_ATOMS
        _wait_expert(aout_ref, eout_ref, sout, e)
        _, dst, w = _router(cls_ref[...], gd_ref[...])
        val = jax.lax.dot_general(
            hid_ref[...], eout_ref[...], (((1,), (1,)), ((), ())),
            preferred_element_type=jnp.float32)
        val = (val + _expert_bias(e, boutb_ref)) * w
        mask = dst == e

        @pl.when(e == 0)
        def _():
            acc_ref[...] = jnp.where(mask, val, 0.0)

        @pl.when(e != 0)
        def _():
            acc_ref[...] = jnp.where(mask, val, acc_ref[...])

        @pl.when(e < _NUM_ATOMS - 1)
        def _():
            _start_expert(aout_ref, eout_ref, sout, e + 1)

    @pl.when(i >= n_steps - 4)
    def _():
        b = i - (n_steps - 4)
        rows = acc_ref[pl.ds(b * _CP, _CP), :]
        rowmask = jax.lax.broadcasted_iota(jnp.int32, (_CP, 1), 0) < _NCLS
        out_ref[0:_CP, :] = jnp.where(rowmask, rows, res[0:_CP, :])


def _tile_index(i):
    k = 2048 // _TILE
    nc = k - 1
    return jnp.where(i < 4 * nc, (i // nc) * k + (i % nc) + 1,
                     (i - 4 * nc) * k)


@jax.jit
def kernel(x, patch_fc1_w, patch_fc1_b, patch_fc2_w, patch_fc2_b,
           gate_delta, atom_in_w, atom_in_b, atom_out_w, atom_out_b):
    B, S, D = x.shape
    H = patch_fc1_w.shape[0]
    T = B * _CP

    x_flat = x.reshape(B * S, D)
    n_tok = B * S
    grid = n_tok // _TILE

    cls_tok = x[:, :_CP].reshape(T, D)
    gd_tok = jnp.tile(
        jnp.pad(gate_delta, ((0, _CP - _NCLS), (0, 0))), (B, 1))

    out = pl.pallas_call(
        _fused_kernel,
        grid=(grid,),
        in_specs=[
            pl.BlockSpec((_TILE, D), lambda i: (_tile_index(i), 0)),
            pl.BlockSpec((H, D), lambda i: (0, 0)),
            pl.BlockSpec((1, H), lambda i: (0, 0)),
            pl.BlockSpec((D, H), lambda i: (0, 0)),
            pl.BlockSpec((1, D), lambda i: (0, 0)),
            pl.BlockSpec((T, D), lambda i: (0, 0)),
            pl.BlockSpec((T, D), lambda i: (0, 0)),
            pl.BlockSpec(memory_space=pl.ANY),
            pl.BlockSpec((_NUM_ATOMS, H), lambda i: (0, 0)),
            pl.BlockSpec(memory_space=pl.ANY),
            pl.BlockSpec((_NUM_ATOMS, D), lambda i: (0, 0)),
        ],
        out_specs=pl.BlockSpec((_TILE, D), lambda i: (_tile_index(i), 0)),
        out_shape=jax.ShapeDtypeStruct((n_tok, D), x.dtype),
        scratch_shapes=[
            pltpu.VMEM((H, D), jnp.bfloat16),
            pltpu.VMEM((D, H), jnp.bfloat16),
            pltpu.VMEM((H, D), jnp.float32),
            pltpu.VMEM((D, H), jnp.float32),
            pltpu.VMEM((T, H), jnp.float32),
            pltpu.VMEM((T, D), jnp.float32),
            pltpu.SemaphoreType.DMA,
            pltpu.SemaphoreType.DMA,
        ],
    )(x_flat, patch_fc1_w, patch_fc1_b.reshape(1, H),
      patch_fc2_w, patch_fc2_b.reshape(1, D),
      cls_tok, gd_tok, atom_in_w, atom_in_b, atom_out_w, atom_out_b)
    return out.reshape(B, S, D)
